# initial kernel scaffold (unmeasured)
import jax
import jax.numpy as jnp
from jax import lax
from jax.experimental import pallas as pl
from jax.experimental.pallas import tpu as pltpu


def kernel(
    x,
):
    def body(*refs):
        pass

    out_shape = jax.ShapeDtypeStruct(..., jnp.float32)
    return pl.pallas_call(body, out_shape=out_shape)(...)


# baseline (device time: 42830 ns/iter reference)
import jax
import jax.numpy as jnp
from jax import lax
from jax.experimental import pallas as pl
from jax.experimental.pallas import tpu as pltpu

N_DEV = 8


def kernel(x):
    m_rows, n_cols = x.shape

    def body(x_ref, out_ref, e_ref, stats_ref, send_sems, recv_sems):
        my = lax.axis_index("i")

        xv = x_ref[...]
        loc_max = jnp.max(xv, axis=1, keepdims=True)
        e = jnp.exp(xv - loc_max)
        loc_sum = jnp.sum(e, axis=1, keepdims=True)
        e_ref[...] = e
        stats_ref[0] = jnp.concatenate([loc_max, loc_sum], axis=1)

        barrier_sem = pltpu.get_barrier_semaphore()
        for d in range(1, N_DEV):
            pl.semaphore_signal(
                barrier_sem, inc=1,
                device_id=((my + d) % N_DEV,),
                device_id_type=pl.DeviceIdType.MESH,
            )
        pl.semaphore_wait(barrier_sem, N_DEV - 1)

        rdmas = []
        for d in range(1, N_DEV):
            rdma = pltpu.make_async_remote_copy(
                src_ref=stats_ref.at[0],
                dst_ref=stats_ref.at[d],
                send_sem=send_sems.at[d],
                recv_sem=recv_sems.at[d],
                device_id=((my + d) % N_DEV,),
                device_id_type=pl.DeviceIdType.MESH,
            )
            rdma.start()
            rdmas.append(rdma)
        for rdma in rdmas:
            rdma.wait_recv()
        for rdma in rdmas:
            rdma.wait_send()

        all_stats = stats_ref[...]
        maxes = all_stats[:, :, 0:1]
        sums = all_stats[:, :, 1:2]
        gmax = jnp.max(maxes, axis=0)
        gsum = jnp.sum(sums * jnp.exp(maxes - gmax), axis=0)
        out_ref[...] = e_ref[...] * (jnp.exp(loc_max - gmax) / gsum)

    return pl.pallas_call(
        body,
        out_shape=jax.ShapeDtypeStruct((m_rows, n_cols), jnp.float32),
        in_specs=[pl.BlockSpec(memory_space=pltpu.VMEM)],
        out_specs=pl.BlockSpec(memory_space=pltpu.VMEM),
        scratch_shapes=[
            pltpu.VMEM((m_rows, n_cols), jnp.float32),
            pltpu.VMEM((N_DEV, m_rows, 2), jnp.float32),
            pltpu.SemaphoreType.DMA((N_DEV,)),
            pltpu.SemaphoreType.DMA((N_DEV,)),
        ],
        compiler_params=pltpu.CompilerParams(collective_id=0),
    )(x)


# device time: 10346 ns/iter; 4.1398x vs baseline; 4.1398x over previous
import jax
import jax.numpy as jnp
from jax import lax
from jax.experimental import pallas as pl
from jax.experimental.pallas import tpu as pltpu

N_DEV = 8


def kernel(x):
    m_rows, n_cols = x.shape

    def body(x_ref, out_ref, stats_ref, send_sems, recv_sems):
        my = lax.axis_index("i")

        barrier_sem = pltpu.get_barrier_semaphore()
        for d in range(1, N_DEV):
            pl.semaphore_signal(
                barrier_sem, inc=1,
                device_id=((my + d) % N_DEV,),
                device_id_type=pl.DeviceIdType.MESH,
            )

        xv = x_ref[...]
        loc_max = jnp.max(xv, axis=1, keepdims=True)
        e = jnp.exp(xv - loc_max)
        loc_sum = jnp.sum(e, axis=1, keepdims=True)
        stats_ref[0] = jnp.transpose(
            jnp.concatenate([loc_max, loc_sum], axis=1)
        )

        pl.semaphore_wait(barrier_sem, N_DEV - 1)

        rdmas = []
        for d in range(1, N_DEV):
            rdma = pltpu.make_async_remote_copy(
                src_ref=stats_ref.at[0],
                dst_ref=stats_ref.at[d],
                send_sem=send_sems.at[d],
                recv_sem=recv_sems.at[d],
                device_id=((my + d) % N_DEV,),
                device_id_type=pl.DeviceIdType.MESH,
            )
            rdma.start()
            rdmas.append(rdma)
        for rdma in rdmas:
            rdma.wait_recv()
        for rdma in rdmas:
            rdma.wait_send()

        all_stats = stats_ref[...]
        maxes = all_stats[:, 0:1, :]
        sums = all_stats[:, 1:2, :]
        gmax = jnp.max(maxes, axis=0)
        gsum = jnp.sum(sums * jnp.exp(maxes - gmax), axis=0)
        g = jnp.transpose(
            jnp.concatenate([gmax, 1.0 / gsum], axis=0)
        )
        out_ref[...] = jnp.exp(x_ref[...] - g[:, 0:1]) * g[:, 1:2]

    return pl.pallas_call(
        body,
        out_shape=jax.ShapeDtypeStruct((m_rows, n_cols), jnp.float32),
        in_specs=[pl.BlockSpec(memory_space=pltpu.VMEM)],
        out_specs=pl.BlockSpec(memory_space=pltpu.VMEM),
        scratch_shapes=[
            pltpu.VMEM((N_DEV, 2, m_rows), jnp.float32),
            pltpu.SemaphoreType.DMA((N_DEV,)),
            pltpu.SemaphoreType.DMA((N_DEV,)),
        ],
        compiler_params=pltpu.CompilerParams(collective_id=0),
    )(x)


# device time: 10088 ns/iter; 4.2456x vs baseline; 1.0256x over previous
import jax
import jax.numpy as jnp
from jax import lax
from jax.experimental import pallas as pl
from jax.experimental.pallas import tpu as pltpu

N_DEV = 8


def kernel(x):
    m_rows, n_cols = x.shape

    def body(x_ref, out_ref, e_ref, stats_ref, send_sems, recv_sems):
        my = lax.axis_index("i")

        barrier_sem = pltpu.get_barrier_semaphore()
        for d in range(1, N_DEV):
            pl.semaphore_signal(
                barrier_sem, inc=1,
                device_id=((my + d) % N_DEV,),
                device_id_type=pl.DeviceIdType.MESH,
            )

        xv = x_ref[...]
        loc_max = jnp.max(xv, axis=1, keepdims=True)
        e = jnp.exp(xv - loc_max)
        loc_sum = jnp.sum(e, axis=1, keepdims=True)
        stats_ref[0] = jnp.transpose(
            jnp.concatenate([loc_max, loc_sum], axis=1)
        )

        pl.semaphore_wait(barrier_sem, N_DEV - 1)

        rdmas = []
        for d in range(1, N_DEV):
            rdma = pltpu.make_async_remote_copy(
                src_ref=stats_ref.at[0],
                dst_ref=stats_ref.at[d],
                send_sem=send_sems.at[d],
                recv_sem=recv_sems.at[d],
                device_id=((my + d) % N_DEV,),
                device_id_type=pl.DeviceIdType.MESH,
            )
            rdma.start()
            rdmas.append(rdma)
        e_ref[...] = e
        for rdma in rdmas:
            rdma.wait_recv()
        for rdma in rdmas:
            rdma.wait_send()

        all_stats = stats_ref[...]
        maxes = all_stats[:, 0:1, :]
        sums = all_stats[:, 1:2, :]
        gmax = jnp.max(maxes, axis=0)
        gsum = jnp.sum(sums * jnp.exp(maxes - gmax), axis=0)
        scale_row = jnp.exp(stats_ref[0, 0:1, :] - gmax) / gsum
        out_ref[...] = e_ref[...] * jnp.transpose(scale_row)

    return pl.pallas_call(
        body,
        out_shape=jax.ShapeDtypeStruct((m_rows, n_cols), jnp.float32),
        in_specs=[pl.BlockSpec(memory_space=pltpu.VMEM)],
        out_specs=pl.BlockSpec(memory_space=pltpu.VMEM),
        scratch_shapes=[
            pltpu.VMEM((m_rows, n_cols), jnp.float32),
            pltpu.VMEM((N_DEV, 2, m_rows), jnp.float32),
            pltpu.SemaphoreType.DMA((N_DEV,)),
            pltpu.SemaphoreType.DMA((N_DEV,)),
        ],
        compiler_params=pltpu.CompilerParams(collective_id=0),
    )(x)
